# baseline (speedup 1.0000x reference)
import functools

import jax
import jax.numpy as jnp
from jax import lax
from jax.experimental import pallas as pl
from jax.experimental.pallas import tpu as pltpu
from jax.experimental.pallas import tpu_sc as plsc

_B, _L, _A = 4096, 2048, 20
_AP = 21
_SKEW = 17
_NC, _NS, _LANES = 2, 16, 16
_NW = _NC * _NS

_B_SC = 2560
_B_TC = _B - _B_SC
_BB = 256

_ROWS_PER_W = _B_SC // _NW
_GROUP = _LANES
_NGROUPS = _ROWS_PER_W // _GROUP
_CHUNKS = _L // _LANES


def _tc_gather_body(x_ref, tbl_ref, out_ref, c_ref):
    xb = x_ref[...]
    cols = [tbl_ref[a, :].reshape(1, _L) for a in range(_A)]

    m0 = (xb & 1) != 0
    m1 = (xb & 2) != 0
    m2 = (xb & 4) != 0
    m3 = (xb & 8) != 0
    m4 = (xb & 16) != 0

    u = [jnp.where(m0, cols[2 * i + 1], cols[2 * i]) for i in range(10)]
    v = [jnp.where(m1, u[2 * i + 1], u[2 * i]) for i in range(5)]
    w0 = jnp.where(m2, v[1], v[0])
    w1 = jnp.where(m2, v[3], v[2])
    z0 = jnp.where(m3, w1, w0)
    g = jnp.where(m4, v[4], z0)
    out_ref[...] = jnp.sum(g, axis=1, keepdims=True)

    @pl.when(pl.program_id(0) == 0)
    def _():
        tb = tbl_ref[...]
        m = jnp.max(tb, axis=0, keepdims=True)
        s = jnp.sum(jnp.exp(tb - m), axis=0, keepdims=True)
        c_ref[...] = jnp.sum(m + jnp.log(s)).reshape(1, 1)


_tc_gather_sum = pl.pallas_call(
    _tc_gather_body,
    grid=(_B_TC // _BB,),
    in_specs=[
        pl.BlockSpec((_BB, _L), lambda i: (i, 0)),
        pl.BlockSpec((_A, _L), lambda i: (0, 0)),
    ],
    out_specs=[
        pl.BlockSpec((_BB, 1), lambda i: (i, 0)),
        pl.BlockSpec((1, 1), lambda i: (0, 0)),
    ],
    out_shape=[
        jax.ShapeDtypeStruct((_B_TC, 1), jnp.float32),
        jax.ShapeDtypeStruct((1, 1), jnp.float32),
    ],
)


def _gather_sum_body(x_hbm, table_hbm, out_hbm, table_v, xbuf, out_v,
                     tr_v, sems, tsem):
    cid = lax.axis_index("c")
    sid = lax.axis_index("s")
    wid = sid * _NC + cid
    row0 = wid * _ROWS_PER_W

    tcopy = pltpu.async_copy(table_hbm, table_v, tsem)

    def start_copy(g, slot):
        return pltpu.async_copy(
            x_hbm.at[pl.ds(_B_TC + row0 + g * _GROUP, _GROUP), :],
            xbuf.at[slot], sems.at[slot])

    start_copy(0, 0)
    lane = lax.iota(jnp.int32, _LANES)
    zero = jnp.zeros((_LANES,), jnp.float32)
    tcopy.wait()

    def group_body(g, _):
        slot = lax.rem(g, 2)

        @pl.when(g + 1 < _NGROUPS)
        def _():
            start_copy(g + 1, lax.rem(g + 1, 2))

        pltpu.make_async_copy(
            x_hbm.at[pl.ds(_B_TC + row0 + g * _GROUP, _GROUP), :],
            xbuf.at[slot], sems.at[slot]).wait()

        @plsc.parallel_loop(0, _CHUNKS, carry=(zero,) * _GROUP)
        def accs(j, carry):
            lvec = lane + j * _LANES
            out = []
            for r in range(_GROUP):
                xv = xbuf[slot, r, pl.ds(j * _LANES, _LANES)]
                t = plsc.load_gather(table_v, [xv, lvec])
                out.append(carry[r] + t)
            return tuple(out)

        for r in range(_GROUP):
            plsc.store_scatter(tr_v, [lane + r * _SKEW], accs[r])
        res = zero
        for i in range(_LANES):
            res = res + plsc.load_gather(tr_v, [lane * _SKEW + i])
        out_v[pl.ds(g * _GROUP, _GROUP)] = res
        return 0

    lax.fori_loop(0, _NGROUPS, group_body, 0)
    pltpu.sync_copy(out_v, out_hbm.at[pl.ds(row0, _ROWS_PER_W)])


_gather_sum = pl.kernel(
    _gather_sum_body,
    out_type=jax.ShapeDtypeStruct((_B_SC,), jnp.float32),
    mesh=plsc.VectorSubcoreMesh(core_axis_name="c", subcore_axis_name="s"),
    compiler_params=pltpu.CompilerParams(needs_layout_passes=False),
    scratch_types=[
        pltpu.VMEM((_A, _L), jnp.float32),
        pltpu.VMEM((2, _GROUP, _L), jnp.int32),
        pltpu.VMEM((_ROWS_PER_W,), jnp.float32),
        pltpu.VMEM((_GROUP * _SKEW,), jnp.float32),
        pltpu.SemaphoreType.DMA((2,)),
        pltpu.SemaphoreType.DMA,
    ],
)


def kernel(x, logits):
    tbl_t = logits.T
    raw_sc = _gather_sum(x, tbl_t)
    raw_tc, c = _tc_gather_sum(x, tbl_t)
    return jnp.concatenate([raw_tc[:, 0], raw_sc]) - c[0, 0]


# speedup vs baseline: 1.1894x; 1.1894x over previous
import functools

import jax
import jax.numpy as jnp
from jax import lax
from jax.experimental import pallas as pl
from jax.experimental.pallas import tpu as pltpu
from jax.experimental.pallas import tpu_sc as plsc

_B, _L, _A = 4096, 2048, 20
_AP = 21
_SKEW = 17
_NC, _NS, _LANES = 2, 16, 16
_NW = _NC * _NS

_B_SC = 2560
_B_TC = _B - _B_SC
_BB = 512

_ROWS_PER_W = _B_SC // _NW
_GROUP = _LANES
_NGROUPS = _ROWS_PER_W // _GROUP
_CHUNKS = _L // _LANES


def _tc_gather_body(x_ref, tbl_ref, out_ref, c_ref):
    xb = x_ref[...]
    cols = [tbl_ref[a, :].reshape(1, _L) for a in range(_A)]

    m0 = (xb & 1) != 0
    m1 = (xb & 2) != 0
    m2 = (xb & 4) != 0
    m3 = (xb & 8) != 0
    m4 = (xb & 16) != 0

    u = [jnp.where(m0, cols[2 * i + 1], cols[2 * i]) for i in range(10)]
    v = [jnp.where(m1, u[2 * i + 1], u[2 * i]) for i in range(5)]
    w0 = jnp.where(m2, v[1], v[0])
    w1 = jnp.where(m2, v[3], v[2])
    z0 = jnp.where(m3, w1, w0)
    g = jnp.where(m4, v[4], z0)
    out_ref[...] = jnp.sum(g, axis=1, keepdims=True)

    @pl.when(pl.program_id(0) == 0)
    def _():
        tb = tbl_ref[...]
        m = jnp.max(tb, axis=0, keepdims=True)
        s = jnp.sum(jnp.exp(tb - m), axis=0, keepdims=True)
        c_ref[...] = jnp.sum(m + jnp.log(s)).reshape(1, 1)


_tc_gather_sum = pl.pallas_call(
    _tc_gather_body,
    grid=(_B_TC // _BB,),
    in_specs=[
        pl.BlockSpec((_BB, _L), lambda i: (i, 0)),
        pl.BlockSpec((_A, _L), lambda i: (0, 0)),
    ],
    out_specs=[
        pl.BlockSpec((_BB, 1), lambda i: (i, 0)),
        pl.BlockSpec((1, 1), lambda i: (0, 0)),
    ],
    out_shape=[
        jax.ShapeDtypeStruct((_B_TC, 1), jnp.float32),
        jax.ShapeDtypeStruct((1, 1), jnp.float32),
    ],
)


def _gather_sum_body(x_hbm, table_hbm, out_hbm, table_v, xbuf, out_v,
                     tr_v, sems, tsem):
    cid = lax.axis_index("c")
    sid = lax.axis_index("s")
    wid = sid * _NC + cid
    row0 = wid * _ROWS_PER_W

    tcopy = pltpu.async_copy(table_hbm, table_v, tsem)

    def start_copy(g, slot):
        return pltpu.async_copy(
            x_hbm.at[pl.ds(_B_TC + row0 + g * _GROUP, _GROUP), :],
            xbuf.at[slot], sems.at[slot])

    start_copy(0, 0)
    lane = lax.iota(jnp.int32, _LANES)
    zero = jnp.zeros((_LANES,), jnp.float32)
    tcopy.wait()

    def group_body(g, _):
        slot = lax.rem(g, 2)

        @pl.when(g + 1 < _NGROUPS)
        def _():
            start_copy(g + 1, lax.rem(g + 1, 2))

        pltpu.make_async_copy(
            x_hbm.at[pl.ds(_B_TC + row0 + g * _GROUP, _GROUP), :],
            xbuf.at[slot], sems.at[slot]).wait()

        @plsc.parallel_loop(0, _CHUNKS, carry=(zero,) * _GROUP)
        def accs(j, carry):
            lvec = lane + j * _LANES
            out = []
            for r in range(_GROUP):
                xv = xbuf[slot, r, pl.ds(j * _LANES, _LANES)]
                t = plsc.load_gather(table_v, [(xv << 11) | lvec])
                out.append(carry[r] + t)
            return tuple(out)

        for r in range(_GROUP):
            plsc.store_scatter(tr_v, [lane + r * _SKEW], accs[r])
        res = zero
        for i in range(_LANES):
            res = res + plsc.load_gather(tr_v, [lane * _SKEW + i])
        out_v[pl.ds(g * _GROUP, _GROUP)] = res
        return 0

    lax.fori_loop(0, _NGROUPS, group_body, 0)
    pltpu.sync_copy(out_v, out_hbm.at[pl.ds(row0, _ROWS_PER_W)])


_gather_sum = pl.kernel(
    _gather_sum_body,
    out_type=jax.ShapeDtypeStruct((_B_SC,), jnp.float32),
    mesh=plsc.VectorSubcoreMesh(core_axis_name="c", subcore_axis_name="s"),
    compiler_params=pltpu.CompilerParams(needs_layout_passes=False),
    scratch_types=[
        pltpu.VMEM((_A * _L,), jnp.float32),
        pltpu.VMEM((2, _GROUP, _L), jnp.int32),
        pltpu.VMEM((_ROWS_PER_W,), jnp.float32),
        pltpu.VMEM((_GROUP * _SKEW,), jnp.float32),
        pltpu.SemaphoreType.DMA((2,)),
        pltpu.SemaphoreType.DMA,
    ],
)


def kernel(x, logits):
    tbl_t = logits.T
    raw_sc = _gather_sum(x, tbl_t.reshape(-1))
    raw_tc, c = _tc_gather_sum(x, tbl_t)
    return jnp.concatenate([raw_tc[:, 0], raw_sc]) - c[0, 0]
